# Initial kernel scaffold; baseline (speedup 1.0000x reference)
import jax
import jax.numpy as jnp
from jax.experimental import pallas as pl


def kernel(data_num, row_num, col_num, cnt_num, data_den, row_den, col_den, cnt_den, weights):
    raise NotImplementedError("write your pallas kernel here")


# speedup vs baseline: 210.4477x; 210.4477x over previous
import functools

import jax
import jax.numpy as jnp
from jax import lax
from jax.experimental import pallas as pl
from jax.experimental.pallas import tpu as pltpu
from jax.experimental.pallas import tpu_sc as plsc

NC = 2
NS = 16
NW = NC * NS
LANES = 16
C = 4096
SCW = 128
NT = C // SCW


def _sc_scatter_call(data_num, col2_num, row2_num, data_den, col2_den,
                     row2_den, weights, r_num, r_den):
    nnz_num = data_num.shape[0]
    nnz_den = data_den.shape[0]
    f = weights.shape[0]

    mesh = plsc.VectorSubcoreMesh(core_axis_name="c", subcore_axis_name="s",
                                  num_cores=NC, num_subcores=NS)

    @functools.partial(
        pl.kernel,
        out_type=[jax.ShapeDtypeStruct((NC * r_num,), jnp.float32),
                  jax.ShapeDtypeStruct((NC * r_den,), jnp.float32)],
        mesh=mesh,
        compiler_params=pltpu.CompilerParams(needs_layout_passes=False),
        scratch_types=[
            pltpu.VMEM((NT, SCW), jnp.int32),
            pltpu.VMEM((C,), jnp.float32),
            pltpu.VMEM((C,), jnp.float32),
            pltpu.VMEM((C,), jnp.float32),
            pltpu.VMEM((NT, SCW), jnp.int32),
            pltpu.VMEM_SHARED((f,), jnp.float32),
            pltpu.VMEM_SHARED((r_num,), jnp.float32),
            pltpu.VMEM_SHARED((r_den,), jnp.float32),
            pltpu.SemaphoreType.DMA,
        ],
    )
    def sc_kernel(dn_hbm, cn_hbm, rn_hbm, dd_hbm, cd_hbm, rd_hbm, w_hbm,
                  out_num, out_den, col_v, dat_v, wg_v, c_v, row_v,
                  w_sh, s_num_sh, s_den_sh, sem):
        cid = lax.axis_index("c")
        sid = lax.axis_index("s")
        wid = cid * NS + sid

        zeros16 = jnp.zeros((LANES,), jnp.float32)

        def zbody(i, _):
            c_v[pl.ds(i * LANES, LANES)] = zeros16
            return 0
        lax.fori_loop(0, C // LANES, zbody, 0)

        def znum(i, _):
            pltpu.sync_copy(
                c_v, s_num_sh.at[pl.ds((sid * (r_num // NS // C) + i) * C, C)])
            return 0
        lax.fori_loop(0, r_num // NS // C, znum, 0)

        def zden(i, _):
            pltpu.sync_copy(
                c_v, s_den_sh.at[pl.ds((sid * (r_den // NS // C) + i) * C, C)])
            return 0
        lax.fori_loop(0, r_den // NS // C, zden, 0)

        @pl.when(sid == 0)
        def _():
            pltpu.sync_copy(w_hbm, w_sh)

        plsc.subcore_barrier()

        def process(col_hbm, dat_hbm, row_hbm, s_sh, nnz):
            per_tile = nnz // NW
            base0 = wid * per_tile

            def chunk(k, _):
                base = pl.multiple_of(base0 + k * C, C)
                b2 = pl.multiple_of(base // SCW, 8)
                pltpu.sync_copy(col_hbm.at[pl.ds(b2, NT)], col_v)
                pltpu.sync_copy(dat_hbm.at[pl.ds(base, C)], dat_v)
                pltpu.sync_copy(row_hbm.at[pl.ds(b2, NT)], row_v)

                gds = []
                for j in range(NT):
                    gds.append(pltpu.async_copy(
                        w_sh.at[col_v.at[j]],
                        wg_v.at[pl.ds(j * SCW, SCW)], sem))
                for d in gds:
                    d.wait()

                def vec(i, _):
                    sl = pl.ds(i * LANES, LANES)
                    c_v[sl] = wg_v[sl] * dat_v[sl]
                    return 0
                lax.fori_loop(0, C // LANES, vec, 0)

                sds = []
                for j in range(NT):
                    sds.append(pltpu.async_copy(
                        c_v.at[pl.ds(j * SCW, SCW)],
                        s_sh.at[row_v.at[j]], sem, add=True))
                for d in sds:
                    d.wait()
                return 0
            lax.fori_loop(0, per_tile // C, chunk, 0)

        process(cd_hbm, dd_hbm, rd_hbm, s_den_sh, nnz_den)
        process(cn_hbm, dn_hbm, rn_hbm, s_num_sh, nnz_num)

        plsc.subcore_barrier()

        ln = r_num // NS
        pltpu.sync_copy(s_num_sh.at[pl.ds(sid * ln, ln)],
                        out_num.at[pl.ds(cid * r_num + sid * ln, ln)])
        ld = r_den // NS
        pltpu.sync_copy(s_den_sh.at[pl.ds(sid * ld, ld)],
                        out_den.at[pl.ds(cid * r_den + sid * ld, ld)])

    return sc_kernel(data_num, col2_num, row2_num, data_den, col2_den,
                     row2_den, weights)


def _tc_finish_body(spn_ref, spd_ref, cn_ref, cd_ref, out_ref):
    i = pl.program_id(0)
    ns = spn_ref[0] + spn_ref[1]
    dsv = spd_ref[0] + spd_ref[1]
    nsum = jnp.sum(jnp.exp(ns) * cn_ref[...], axis=1, keepdims=True)
    dsum = jnp.sum(jnp.exp(dsv) * cd_ref[...], axis=1, keepdims=True)
    part = (jnp.sum(jnp.log(dsum), keepdims=True)
            - jnp.sum(jnp.log(nsum), keepdims=True))

    @pl.when(i == 0)
    def _():
        out_ref[...] = jnp.zeros_like(out_ref)
    out_ref[...] += part


def _tc_finish(sp_num3, sp_den3, cnt_num2, cnt_den2):
    n, mr_num = cnt_num2.shape
    mr_den = cnt_den2.shape[1]
    rb = 1024
    grid = (n // rb,)
    return pl.pallas_call(
        _tc_finish_body,
        grid=grid,
        in_specs=[
            pl.BlockSpec((NC, rb, mr_num), lambda i: (0, i, 0)),
            pl.BlockSpec((NC, rb, mr_den), lambda i: (0, i, 0)),
            pl.BlockSpec((rb, mr_num), lambda i: (i, 0)),
            pl.BlockSpec((rb, mr_den), lambda i: (i, 0)),
        ],
        out_specs=pl.BlockSpec((1, 1), lambda i: (0, 0)),
        out_shape=jax.ShapeDtypeStruct((1, 1), jnp.float32),
    )(sp_num3, sp_den3, cnt_num2, cnt_den2)


def kernel(data_num, row_num, col_num, cnt_num, data_den, row_den, col_den,
           cnt_den, weights):
    r_num = cnt_num.shape[0]
    r_den = cnt_den.shape[0]
    col2_num = col_num.reshape(-1, SCW)
    col2_den = col_den.reshape(-1, SCW)
    row2_num = row_num.reshape(-1, SCW)
    row2_den = row_den.reshape(-1, SCW)
    sp_num, sp_den = _sc_scatter_call(data_num, col2_num, row2_num,
                                      data_den, col2_den, row2_den,
                                      weights, r_num, r_den)
    n = 16384
    sp_num3 = sp_num.reshape(NC, n, r_num // n)
    sp_den3 = sp_den.reshape(NC, n, r_den // n)
    cnt_num2 = cnt_num.reshape(n, r_num // n)
    cnt_den2 = cnt_den.reshape(n, r_den // n)
    loss = _tc_finish(sp_num3, sp_den3, cnt_num2, cnt_den2)
    return loss[0, 0]
